# Initial kernel scaffold; baseline (speedup 1.0000x reference)
import jax
import jax.numpy as jnp
from jax.experimental import pallas as pl


def kernel(inputs, targets, margin):
    raise NotImplementedError("write your pallas kernel here")


# speedup vs baseline: 1213.9420x; 1213.9420x over previous
import jax
import jax.numpy as jnp
from jax import lax
from jax.experimental import pallas as pl
from jax.experimental.pallas import tpu as pltpu

_ALPHA = 10.0
_BETA = 2.0
_BASE = 0.7


def _scl_kernel(x_ref, tcol_ref, trow_ref, m_ref, out_ref):
    n = x_ref.shape[0]
    x = x_ref[...]
    sim = lax.dot_general(x, x, (((1,), (1,)), ((), ())),
                          preferred_element_type=jnp.float32)
    tcol = tcol_ref[...]
    trow = trow_ref[...]
    m = m_ref[0, 0]

    pmask = tcol == trow
    pos_valid = pmask & (sim < 1.0 - 1e-06)
    neg_valid = jnp.logical_not(pmask)
    posf = pos_valid.astype(jnp.float32)
    negf = neg_valid.astype(jnp.float32)

    cnt_p = jnp.sum(posf, axis=1, keepdims=True)
    cnt_n = jnp.sum(negf, axis=1, keepdims=True)
    inf = jnp.float32(jnp.inf)
    pos0 = jnp.min(jnp.where(pos_valid, sim, inf), axis=1, keepdims=True)
    negmax = jnp.max(jnp.where(neg_valid, sim, -inf), axis=1, keepdims=True)

    t_n = jax.nn.relu((sim + m) - pos0)
    t_p = jax.nn.relu((negmax - sim) + m)
    mask_n = neg_valid & (t_n > 0)
    mask_p = pos_valid & (t_p > 0)
    val_n = (t_n - m) + pos0
    val_p = (-t_p + m) + negmax

    count_an = jnp.sum(mask_n.astype(jnp.float32), axis=1, keepdims=True)
    count_ap = jnp.sum(mask_p.astype(jnp.float32), axis=1, keepdims=True)

    pos_terms = jnp.log(1.0 + jnp.exp(-_BETA * (val_p - _BASE)))
    neg_terms = jnp.log(1.0 + jnp.exp(_ALPHA * (val_n - _BASE)))
    pos_sum = jnp.sum(jnp.where(mask_p, pos_terms, 0.0), axis=1, keepdims=True)
    neg_sum = jnp.sum(jnp.where(mask_n, neg_terms, 0.0), axis=1, keepdims=True)
    pos_loss = 2.0 / _BETA * (pos_sum / jnp.maximum(count_ap, 1.0))
    neg_loss = 2.0 / _ALPHA * (neg_sum / jnp.maximum(count_an, 1.0))
    valid = (count_an >= 1.0) & (count_ap >= 1.0)

    loss_acc = jnp.sum(jnp.where(valid, neg_loss + pos_loss, 0.0))
    c = jnp.sum(jnp.where(valid, 0.0, 1.0))
    length_ap = jnp.sum(jnp.where(valid, count_ap, 0.0))
    length_an = jnp.sum(jnp.where(valid, count_an, 0.0))
    length_big_p = jnp.sum(cnt_p)

    sim_last = sim[n - 1:n, :]
    pv_last = pos_valid[n - 1:n, :]
    nv_last = neg_valid[n - 1:n, :]
    sum_pos_last = jnp.sum(jnp.where(pv_last, sim_last, 0.0))
    sum_neg_last = jnp.sum(jnp.where(nv_last, sim_last, 0.0))
    mean_pos = sum_pos_last / cnt_p[n - 1, 0]
    mean_neg = sum_neg_last / cnt_n[n - 1, 0]

    loss = loss_acc / n
    prec = c / n
    lp = length_ap / 2.0
    ln_ = length_an / 2.0
    big_lp = length_big_p / 2.0
    r1 = ln_ / lp
    s1 = 1.0 / (1.0 + jnp.exp(-r1))
    r2 = ln_ / big_lp
    s2 = 1.0 / (1.0 + jnp.exp(-r2))
    r3 = lp / big_lp
    zero = jnp.float32(0.0)
    vals = jnp.stack([loss, prec, mean_pos, mean_neg, lp, ln_,
                      r1, s1, r2, s2, r3, zero, zero, zero, zero, zero])
    out_ref[...] = vals[None, :]


def kernel(inputs, targets, margin):
    n = inputs.shape[0]
    x = inputs.astype(jnp.float32)
    t = targets.astype(jnp.int32)
    tcol = t.reshape(n, 1)
    trow = t.reshape(1, n)
    m = jnp.asarray(margin, jnp.float32).reshape(1, 1)

    out = pl.pallas_call(
        _scl_kernel,
        out_shape=jax.ShapeDtypeStruct((1, 16), jnp.float32),
    )(x, tcol, trow, m)

    v = out[0]
    stats = v[6:11]
    return (v[0], v[1], v[2], v[3], v[4], v[5], stats, stats)


# speedup vs baseline: 1216.5802x; 1.0022x over previous
import jax
import jax.numpy as jnp
from jax import lax
from jax.experimental import pallas as pl
from jax.experimental.pallas import tpu as pltpu

_ALPHA = 10.0
_BETA = 2.0
_BASE = 0.7


def _scl_kernel(x_ref, tcol_ref, trow_ref, m_ref, out_ref):
    n = x_ref.shape[0]
    x = x_ref[...]
    sim = lax.dot_general(x, x, (((1,), (1,)), ((), ())),
                          preferred_element_type=jnp.float32)
    tcol = tcol_ref[...]
    trow = trow_ref[...]
    m = m_ref[0, 0]

    pmask = tcol == trow
    pos_valid = pmask & (sim < 1.0 - 1e-06)
    neg_valid = jnp.logical_not(pmask)
    posf = pos_valid.astype(jnp.float32)
    negf = neg_valid.astype(jnp.float32)

    cnt_p = jnp.sum(posf, axis=1, keepdims=True)
    cnt_n = jnp.sum(negf, axis=1, keepdims=True)
    inf = jnp.float32(jnp.inf)
    pos0 = jnp.min(jnp.where(pos_valid, sim, inf), axis=1, keepdims=True)
    negmax = jnp.max(jnp.where(neg_valid, sim, -inf), axis=1, keepdims=True)

    t = jax.nn.relu(jnp.where(pmask, (negmax - sim) + m, (sim + m) - pos0))
    mask = (neg_valid | (sim < 1.0 - 1e-06)) & (t > 0)
    mask_p = pmask & mask
    mask_n = (~pmask) & mask
    val = jnp.where(pmask, (-t + m) + negmax, (t - m) + pos0)
    arg = jnp.where(pmask, -_BETA * (val - _BASE), _ALPHA * (val - _BASE))
    term = jnp.log(1.0 + jnp.exp(arg))

    count_an = jnp.sum(mask_n.astype(jnp.float32), axis=1, keepdims=True)
    count_ap = jnp.sum(mask_p.astype(jnp.float32), axis=1, keepdims=True)
    pos_sum = jnp.sum(jnp.where(mask_p, term, 0.0), axis=1, keepdims=True)
    neg_sum = jnp.sum(jnp.where(mask_n, term, 0.0), axis=1, keepdims=True)
    pos_loss = 2.0 / _BETA * (pos_sum / jnp.maximum(count_ap, 1.0))
    neg_loss = 2.0 / _ALPHA * (neg_sum / jnp.maximum(count_an, 1.0))
    valid = (count_an >= 1.0) & (count_ap >= 1.0)

    loss_acc = jnp.sum(jnp.where(valid, neg_loss + pos_loss, 0.0))
    c = jnp.sum(jnp.where(valid, 0.0, 1.0))
    length_ap = jnp.sum(jnp.where(valid, count_ap, 0.0))
    length_an = jnp.sum(jnp.where(valid, count_an, 0.0))
    length_big_p = jnp.sum(cnt_p)

    sim_last = sim[n - 1:n, :]
    pv_last = pos_valid[n - 1:n, :]
    nv_last = neg_valid[n - 1:n, :]
    sum_pos_last = jnp.sum(jnp.where(pv_last, sim_last, 0.0))
    sum_neg_last = jnp.sum(jnp.where(nv_last, sim_last, 0.0))
    mean_pos = sum_pos_last / cnt_p[n - 1, 0]
    mean_neg = sum_neg_last / cnt_n[n - 1, 0]

    loss = loss_acc / n
    prec = c / n
    lp = length_ap / 2.0
    ln_ = length_an / 2.0
    big_lp = length_big_p / 2.0
    r1 = ln_ / lp
    s1 = 1.0 / (1.0 + jnp.exp(-r1))
    r2 = ln_ / big_lp
    s2 = 1.0 / (1.0 + jnp.exp(-r2))
    r3 = lp / big_lp
    zero = jnp.float32(0.0)
    vals = jnp.stack([loss, prec, mean_pos, mean_neg, lp, ln_,
                      r1, s1, r2, s2, r3, zero, zero, zero, zero, zero])
    out_ref[...] = vals[None, :]


def kernel(inputs, targets, margin):
    n = inputs.shape[0]
    x = inputs.astype(jnp.float32)
    t = targets.astype(jnp.int32)
    tcol = t.reshape(n, 1)
    trow = t.reshape(1, n)
    m = jnp.asarray(margin, jnp.float32).reshape(1, 1)

    out = pl.pallas_call(
        _scl_kernel,
        out_shape=jax.ShapeDtypeStruct((1, 16), jnp.float32),
    )(x, tcol, trow, m)

    v = out[0]
    stats = v[6:11]
    return (v[0], v[1], v[2], v[3], v[4], v[5], stats, stats)


# speedup vs baseline: 1310.4101x; 1.0771x over previous
import jax
import jax.numpy as jnp
from jax import lax
from jax.experimental import pallas as pl
from jax.experimental.pallas import tpu as pltpu

_ALPHA = 10.0
_BETA = 2.0
_BASE = 0.7


def _scl_kernel(x_ref, tcol_ref, trow_ref, m_ref,
                loss_ref, prec_ref, mpos_ref, mneg_ref, lp_ref, ln_ref,
                stats_ref):
    n = x_ref.shape[0]
    x = x_ref[...]
    sim = lax.dot_general(x, x, (((1,), (1,)), ((), ())),
                          preferred_element_type=jnp.float32)
    tcol = tcol_ref[...]
    trow = trow_ref[...]
    m = m_ref[0, 0]

    pmask = tcol == trow
    pos_valid = pmask & (sim < 1.0 - 1e-06)
    neg_valid = jnp.logical_not(pmask)
    posf = pos_valid.astype(jnp.float32)
    negf = neg_valid.astype(jnp.float32)

    cnt_p = jnp.sum(posf, axis=1, keepdims=True)
    cnt_n = jnp.sum(negf, axis=1, keepdims=True)
    inf = jnp.float32(jnp.inf)
    pos0 = jnp.min(jnp.where(pos_valid, sim, inf), axis=1, keepdims=True)
    negmax = jnp.max(jnp.where(neg_valid, sim, -inf), axis=1, keepdims=True)

    t = jax.nn.relu(jnp.where(pmask, (negmax - sim) + m, (sim + m) - pos0))
    mask = (neg_valid | (sim < 1.0 - 1e-06)) & (t > 0)
    mask_p = pmask & mask
    mask_n = (~pmask) & mask
    val = jnp.where(pmask, (-t + m) + negmax, (t - m) + pos0)
    arg = jnp.where(pmask, -_BETA * (val - _BASE), _ALPHA * (val - _BASE))
    term = jnp.log(1.0 + jnp.exp(arg))

    count_an = jnp.sum(mask_n.astype(jnp.float32), axis=1, keepdims=True)
    count_ap = jnp.sum(mask_p.astype(jnp.float32), axis=1, keepdims=True)
    pos_sum = jnp.sum(jnp.where(mask_p, term, 0.0), axis=1, keepdims=True)
    neg_sum = jnp.sum(jnp.where(mask_n, term, 0.0), axis=1, keepdims=True)
    pos_loss = 2.0 / _BETA * (pos_sum / jnp.maximum(count_ap, 1.0))
    neg_loss = 2.0 / _ALPHA * (neg_sum / jnp.maximum(count_an, 1.0))
    valid = (count_an >= 1.0) & (count_ap >= 1.0)

    loss_acc = jnp.sum(jnp.where(valid, neg_loss + pos_loss, 0.0))
    c = jnp.sum(jnp.where(valid, 0.0, 1.0))
    length_ap = jnp.sum(jnp.where(valid, count_ap, 0.0))
    length_an = jnp.sum(jnp.where(valid, count_an, 0.0))
    length_big_p = jnp.sum(cnt_p)

    sim_last = sim[n - 1:n, :]
    pv_last = pos_valid[n - 1:n, :]
    nv_last = neg_valid[n - 1:n, :]
    sum_pos_last = jnp.sum(jnp.where(pv_last, sim_last, 0.0))
    sum_neg_last = jnp.sum(jnp.where(nv_last, sim_last, 0.0))
    mean_pos = sum_pos_last / cnt_p[n - 1, 0]
    mean_neg = sum_neg_last / cnt_n[n - 1, 0]

    loss = loss_acc / n
    prec = c / n
    lp = length_ap / 2.0
    ln_ = length_an / 2.0
    big_lp = length_big_p / 2.0
    r1 = ln_ / lp
    s1 = 1.0 / (1.0 + jnp.exp(-r1))
    r2 = ln_ / big_lp
    s2 = 1.0 / (1.0 + jnp.exp(-r2))
    r3 = lp / big_lp
    loss_ref[0] = loss
    prec_ref[0] = prec
    mpos_ref[0] = mean_pos
    mneg_ref[0] = mean_neg
    lp_ref[0] = lp
    ln_ref[0] = ln_
    stats_ref[0] = r1
    stats_ref[1] = s1
    stats_ref[2] = r2
    stats_ref[3] = s2
    stats_ref[4] = r3


def kernel(inputs, targets, margin):
    n = inputs.shape[0]
    x = inputs.astype(jnp.float32)
    t = targets.astype(jnp.int32)
    tcol = t.reshape(n, 1)
    trow = t.reshape(1, n)
    m = jnp.asarray(margin, jnp.float32).reshape(1, 1)

    smem = pl.BlockSpec(memory_space=pltpu.SMEM)
    scalar = jax.ShapeDtypeStruct((1,), jnp.float32)
    outs = pl.pallas_call(
        _scl_kernel,
        out_shape=(scalar, scalar, scalar, scalar, scalar, scalar,
                   jax.ShapeDtypeStruct((5,), jnp.float32)),
        out_specs=(smem, smem, smem, smem, smem, smem, smem),
    )(x, tcol, trow, m)

    loss, prec, mpos, mneg, lp, ln_, stats = outs
    return (loss.reshape(()), prec.reshape(()), mpos.reshape(()),
            mneg.reshape(()), lp.reshape(()), ln_.reshape(()), stats, stats)


# speedup vs baseline: 1612.6507x; 1.2306x over previous
import jax
import jax.numpy as jnp
from jax import lax
from jax.experimental import pallas as pl
from jax.experimental.pallas import tpu as pltpu

_ALPHA = 10.0
_BETA = 2.0
_BASE = 0.7


def _scl_kernel(x_ref, t_ref, m_ref,
                loss_ref, prec_ref, mpos_ref, mneg_ref, lp_ref, ln_ref,
                stats_ref):
    n = x_ref.shape[0]
    x = x_ref[...]
    sim = lax.dot_general(x, x, (((1,), (1,)), ((), ())),
                          preferred_element_type=jnp.float32)
    m = m_ref[0, 0]

    trow = t_ref[...].astype(jnp.float32).reshape(1, n)
    ident = (lax.broadcasted_iota(jnp.int32, (n, n), 0)
             == lax.broadcasted_iota(jnp.int32, (n, n), 1)).astype(jnp.float32)
    tcol = lax.dot_general(ident, trow, (((1,), (1,)), ((), ())),
                           preferred_element_type=jnp.float32)

    ones_col = jnp.ones((n, 1), jnp.float32)

    def row_sum(a):
        return lax.dot_general(a, ones_col, (((1,), (0,)), ((), ())),
                               preferred_element_type=jnp.float32)

    pmask = tcol == trow
    pos_valid = pmask & (sim < 1.0 - 1e-06)
    neg_valid = jnp.logical_not(pmask)
    posf = pos_valid.astype(jnp.float32)
    negf = neg_valid.astype(jnp.float32)

    cnt_p = row_sum(posf)
    cnt_n = row_sum(negf)
    inf = jnp.float32(jnp.inf)
    pos0 = jnp.min(jnp.where(pos_valid, sim, inf), axis=1, keepdims=True)
    negmax = jnp.max(jnp.where(neg_valid, sim, -inf), axis=1, keepdims=True)

    t = jax.nn.relu(jnp.where(pmask, (negmax - sim) + m, (sim + m) - pos0))
    mask = (neg_valid | (sim < 1.0 - 1e-06)) & (t > 0)
    maskf = mask.astype(jnp.float32)
    mask_pf = posf * maskf
    mask_nf = negf * maskf
    val = jnp.where(pmask, (-t + m) + negmax, (t - m) + pos0)
    arg = jnp.where(pmask, -_BETA * (val - _BASE), _ALPHA * (val - _BASE))
    term = jnp.log(1.0 + jnp.exp(arg))

    count_ap = row_sum(mask_pf)
    count_an = row_sum(mask_nf)
    pos_sum = row_sum(mask_pf * term)
    neg_sum = row_sum(mask_nf * term)
    pos_loss = 2.0 / _BETA * (pos_sum / jnp.maximum(count_ap, 1.0))
    neg_loss = 2.0 / _ALPHA * (neg_sum / jnp.maximum(count_an, 1.0))
    valid = (count_an >= 1.0) & (count_ap >= 1.0)

    loss_acc = jnp.sum(jnp.where(valid, neg_loss + pos_loss, 0.0))
    c = jnp.sum(jnp.where(valid, 0.0, 1.0))
    length_ap = jnp.sum(jnp.where(valid, count_ap, 0.0))
    length_an = jnp.sum(jnp.where(valid, count_an, 0.0))
    length_big_p = jnp.sum(cnt_p)

    sim_last = sim[n - 1:n, :]
    pv_last = pos_valid[n - 1:n, :]
    nv_last = neg_valid[n - 1:n, :]
    sum_pos_last = jnp.sum(jnp.where(pv_last, sim_last, 0.0))
    sum_neg_last = jnp.sum(jnp.where(nv_last, sim_last, 0.0))
    mean_pos = sum_pos_last / cnt_p[n - 1, 0]
    mean_neg = sum_neg_last / cnt_n[n - 1, 0]

    loss = loss_acc / n
    prec = c / n
    lp = length_ap / 2.0
    ln_ = length_an / 2.0
    big_lp = length_big_p / 2.0
    r1 = ln_ / lp
    s1 = 1.0 / (1.0 + jnp.exp(-r1))
    r2 = ln_ / big_lp
    s2 = 1.0 / (1.0 + jnp.exp(-r2))
    r3 = lp / big_lp
    loss_ref[0] = loss
    prec_ref[0] = prec
    mpos_ref[0] = mean_pos
    mneg_ref[0] = mean_neg
    lp_ref[0] = lp
    ln_ref[0] = ln_
    stats_ref[0] = r1
    stats_ref[1] = s1
    stats_ref[2] = r2
    stats_ref[3] = s2
    stats_ref[4] = r3


def kernel(inputs, targets, margin):
    x = inputs.astype(jnp.float32)
    t = targets.astype(jnp.int32)
    m = jnp.asarray(margin, jnp.float32).reshape(1, 1)

    smem = pl.BlockSpec(memory_space=pltpu.SMEM)
    scalar = jax.ShapeDtypeStruct((1,), jnp.float32)
    outs = pl.pallas_call(
        _scl_kernel,
        out_shape=(scalar, scalar, scalar, scalar, scalar, scalar,
                   jax.ShapeDtypeStruct((5,), jnp.float32)),
        out_specs=(smem, smem, smem, smem, smem, smem, smem),
    )(x, t, m)

    loss, prec, mpos, mneg, lp, ln_, stats = outs
    return (loss.reshape(()), prec.reshape(()), mpos.reshape(()),
            mneg.reshape(()), lp.reshape(()), ln_.reshape(()), stats, stats)


# speedup vs baseline: 1687.0193x; 1.0461x over previous
import jax
import jax.numpy as jnp
from jax import lax
from jax.experimental import pallas as pl
from jax.experimental.pallas import tpu as pltpu

_ALPHA = 10.0
_BETA = 2.0
_BASE = 0.7


def _scl_kernel(x_ref, t_ref, m_ref,
                loss_ref, prec_ref, mpos_ref, mneg_ref, lp_ref, ln_ref,
                stats_ref):
    n = x_ref.shape[0]
    x = x_ref[...]
    sim = lax.dot_general(x, x, (((1,), (1,)), ((), ())),
                          preferred_element_type=jnp.float32)
    m = m_ref[0, 0]

    trow = t_ref[...].astype(jnp.float32).reshape(1, n)
    ident = (lax.broadcasted_iota(jnp.int32, (n, n), 0)
             == lax.broadcasted_iota(jnp.int32, (n, n), 1)).astype(jnp.float32)
    tcol = lax.dot_general(ident, trow, (((1,), (1,)), ((), ())),
                           preferred_element_type=jnp.float32)

    ones_col = jnp.ones((n, 1), jnp.float32)

    def row_sum(a):
        return lax.dot_general(a, ones_col, (((1,), (0,)), ((), ())),
                               preferred_element_type=jnp.float32)

    pmask = tcol == trow
    pos_valid = pmask & (sim < 1.0 - 1e-06)
    neg_valid = jnp.logical_not(pmask)
    posf = pos_valid.astype(jnp.float32)
    negf = neg_valid.astype(jnp.float32)

    cnt_p = row_sum(posf)
    cnt_n = row_sum(negf)
    inf = jnp.float32(jnp.inf)
    pos0 = jnp.min(jnp.where(pos_valid, sim, inf), axis=1, keepdims=True)
    negmax = jnp.max(jnp.where(neg_valid, sim, -inf), axis=1, keepdims=True)

    cp = negmax + m
    cn = m - pos0
    rp = -_BETA * ((m - _BASE) + negmax)
    rn = _ALPHA * ((pos0 - m) - _BASE)
    t = jax.nn.relu(jnp.where(pmask, cp - sim, sim + cn))
    mask = (neg_valid | (sim < 1.0 - 1e-06)) & (t > 0)
    maskf = mask.astype(jnp.float32)
    mask_pf = posf * maskf
    mask_nf = negf * maskf
    arg = jnp.where(pmask, _BETA, _ALPHA) * t + jnp.where(pmask, rp, rn)
    term = jnp.log(1.0 + jnp.exp(arg))

    count_ap = row_sum(mask_pf)
    count_an = row_sum(mask_nf)
    pos_sum = row_sum(mask_pf * term)
    neg_sum = row_sum(mask_nf * term)
    pos_loss = 2.0 / _BETA * (pos_sum / jnp.maximum(count_ap, 1.0))
    neg_loss = 2.0 / _ALPHA * (neg_sum / jnp.maximum(count_an, 1.0))
    valid = (count_an >= 1.0) & (count_ap >= 1.0)

    cols = jnp.concatenate(
        [jnp.where(valid, neg_loss + pos_loss, 0.0),
         jnp.where(valid, 0.0, 1.0),
         jnp.where(valid, count_ap, 0.0),
         jnp.where(valid, count_an, 0.0),
         cnt_p], axis=1)
    sums = jnp.sum(cols, axis=0)
    loss_acc = sums[0]
    c = sums[1]
    length_ap = sums[2]
    length_an = sums[3]
    length_big_p = sums[4]

    sim_last = sim[n - 1:n, :]
    pv_last = pos_valid[n - 1:n, :]
    nv_last = neg_valid[n - 1:n, :]
    sum_pos_last = jnp.sum(jnp.where(pv_last, sim_last, 0.0))
    sum_neg_last = jnp.sum(jnp.where(nv_last, sim_last, 0.0))
    mean_pos = sum_pos_last / cnt_p[n - 1, 0]
    mean_neg = sum_neg_last / cnt_n[n - 1, 0]

    loss = loss_acc / n
    prec = c / n
    lp = length_ap / 2.0
    ln_ = length_an / 2.0
    big_lp = length_big_p / 2.0
    r1 = ln_ / lp
    s1 = 1.0 / (1.0 + jnp.exp(-r1))
    r2 = ln_ / big_lp
    s2 = 1.0 / (1.0 + jnp.exp(-r2))
    r3 = lp / big_lp
    loss_ref[0] = loss
    prec_ref[0] = prec
    mpos_ref[0] = mean_pos
    mneg_ref[0] = mean_neg
    lp_ref[0] = lp
    ln_ref[0] = ln_
    stats_ref[0] = r1
    stats_ref[1] = s1
    stats_ref[2] = r2
    stats_ref[3] = s2
    stats_ref[4] = r3


def kernel(inputs, targets, margin):
    x = inputs.astype(jnp.float32)
    t = targets.astype(jnp.int32)
    m = jnp.asarray(margin, jnp.float32).reshape(1, 1)

    smem = pl.BlockSpec(memory_space=pltpu.SMEM)
    scalar = jax.ShapeDtypeStruct((1,), jnp.float32)
    outs = pl.pallas_call(
        _scl_kernel,
        out_shape=(scalar, scalar, scalar, scalar, scalar, scalar,
                   jax.ShapeDtypeStruct((5,), jnp.float32)),
        out_specs=(smem, smem, smem, smem, smem, smem, smem),
    )(x, t, m)

    loss, prec, mpos, mneg, lp, ln_, stats = outs
    return (loss.reshape(()), prec.reshape(()), mpos.reshape(()),
            mneg.reshape(()), lp.reshape(()), ln_.reshape(()), stats, stats)
